# baseline (speedup 1.0000x reference)
import functools
import math

import jax
import jax.numpy as jnp
from jax import lax
from jax.experimental import pallas as pl
from jax.experimental.pallas import tpu as pltpu
from jax.experimental.pallas import tpu_sc as plsc

N = 10000
E = 320000
NFEAT = 128
NHID = 128
NCLASS = 64
BN_EPS = 1e-5

NC = 2
NS = 16
NW = NC * NS
CH = 128
NCHUNK = 160
EPT = NCHUNK * CH
E_PAD = NS * EPT
ACC_N = 10240
D0 = 64
D2 = 32
BR = 2048
_BN_SCALE = 1.0 / math.sqrt(1.0 + BN_EPS)


def _make_seg_sum(drow):
  mesh = plsc.VectorSubcoreMesh(core_axis_name="c", subcore_axis_name="s")
  rps = ACC_N // NS

  @functools.partial(
      pl.kernel,
      out_type=jax.ShapeDtypeStruct((NC, ACC_N, drow), jnp.float32),
      mesh=mesh,
      compiler_params=pltpu.CompilerParams(use_tc_tiling_on_sc=False,
                                           needs_layout_passes=False),
      scratch_types=[
          pltpu.VMEM((NCHUNK, CH), jnp.int32),
          pltpu.VMEM((NCHUNK, CH), jnp.int32),
          pltpu.VMEM((CH, drow), jnp.float32),
          pltpu.VMEM((CH, drow), jnp.float32),
          pltpu.VMEM((16, drow), jnp.float32),
          pltpu.VMEM_SHARED((ACC_N, drow), jnp.float32),
          pltpu.SemaphoreType.DMA,
          pltpu.SemaphoreType.DMA,
      ],
  )
  def seg_sum(y_hbm, src_hbm, dst_hbm, out_hbm,
              src_v, dst_v, rows0, rows1, zbuf, acc, sem0, sem1):
    c = lax.axis_index("c")
    s = lax.axis_index("s")

    zv = jnp.zeros((16,), jnp.float32)
    for i in range(16):
      for j in range(drow // 16):
        zbuf[i, pl.ds(j * 16, 16)] = zv

    @pl.loop(0, rps // 16)
    def _zero(k):
      pltpu.sync_copy(zbuf, acc.at[pl.ds(s * rps + k * 16, 16)])

    pltpu.sync_copy(src_hbm.at[c, s], src_v)
    pltpu.sync_copy(dst_hbm.at[s], dst_v)
    plsc.subcore_barrier()

    pltpu.async_copy(y_hbm.at[src_v.at[0]], rows0, sem0)
    pltpu.async_copy(y_hbm.at[src_v.at[1]], rows1, sem1)

    @pl.loop(0, (NCHUNK - 2) // 2)
    def _chunks(j):
      i = 2 * j
      pltpu.make_async_copy(y_hbm.at[src_v.at[i]], rows0, sem0).wait()
      pltpu.sync_copy(rows0, acc.at[dst_v.at[i]], add=True)
      pltpu.async_copy(y_hbm.at[src_v.at[i + 2]], rows0, sem0)
      pltpu.make_async_copy(y_hbm.at[src_v.at[i + 1]], rows1, sem1).wait()
      pltpu.sync_copy(rows1, acc.at[dst_v.at[i + 1]], add=True)
      pltpu.async_copy(y_hbm.at[src_v.at[i + 3]], rows1, sem1)

    pltpu.make_async_copy(y_hbm.at[src_v.at[NCHUNK - 2]], rows0, sem0).wait()
    pltpu.sync_copy(rows0, acc.at[dst_v.at[NCHUNK - 2]], add=True)
    pltpu.make_async_copy(y_hbm.at[src_v.at[NCHUNK - 1]], rows1, sem1).wait()
    pltpu.sync_copy(rows1, acc.at[dst_v.at[NCHUNK - 1]], add=True)

    plsc.subcore_barrier()
    pltpu.sync_copy(acc.at[pl.ds(s * rps, rps)],
                    out_hbm.at[c, pl.ds(s * rps, rps)])

  return seg_sum


_seg_sum_l0 = _make_seg_sum(D0)
_seg_sum_l2 = _make_seg_sum(D2)


def _make_count():
  mesh = plsc.VectorSubcoreMesh(core_axis_name="c", subcore_axis_name="s")
  half = NCHUNK // 2

  @functools.partial(
      pl.kernel,
      out_type=jax.ShapeDtypeStruct((NC * NS, ACC_N), jnp.float32),
      mesh=mesh,
      compiler_params=pltpu.CompilerParams(use_tc_tiling_on_sc=False,
                                           needs_layout_passes=False),
      scratch_types=[
          pltpu.VMEM((half, CH), jnp.int32),
          pltpu.VMEM((ACC_N,), jnp.float32),
      ],
  )
  def count_k(dst_hbm, out_hbm, dst_v, cnt_v):
    c = lax.axis_index("c")
    s = lax.axis_index("s")

    pltpu.sync_copy(dst_hbm.at[s, pl.ds(c * half, half)], dst_v)

    zv = jnp.zeros((16,), jnp.float32)
    for k in range(ACC_N // 16):
      cnt_v[pl.ds(k * 16, 16)] = zv

    ones16 = jnp.ones((16,), jnp.float32)

    @pl.loop(0, half)
    def _chunks(i):
      for k in range(CH // 16):
        idx = dst_v[i, pl.ds(k * 16, 16)]
        plsc.addupdate_scatter(cnt_v, [idx], ones16)

    pltpu.sync_copy(cnt_v, out_hbm.at[c * NS + s])

  return count_k


_count_k = _make_count()


def _stage_a(x, wl0t, wr0t):
  def body(x_ref, wl_ref, wr_ref, ycat_ref, r0_ref):
    xv = x_ref[...]
    y0 = jnp.dot(xv, wl_ref[...], preferred_element_type=jnp.float32)
    ycat_ref[0] = y0[:, :D0]
    ycat_ref[1] = y0[:, D0:]
    r0_ref[...] = jnp.dot(xv, wr_ref[...], preferred_element_type=jnp.float32)

  return pl.pallas_call(
      body,
      grid=(ACC_N // BR,),
      in_specs=[
          pl.BlockSpec((BR, NFEAT), lambda i: (i, 0)),
          pl.BlockSpec((NFEAT, NHID), lambda i: (0, 0)),
          pl.BlockSpec((NFEAT, NHID), lambda i: (0, 0)),
      ],
      out_specs=[
          pl.BlockSpec((NC, BR, D0), lambda i: (0, i, 0)),
          pl.BlockSpec((BR, NHID), lambda i: (i, 0)),
      ],
      out_shape=[
          jax.ShapeDtypeStruct((NC, ACC_N, D0), jnp.float32),
          jax.ShapeDtypeStruct((ACC_N, NHID), jnp.float32),
      ],
  )(x, wl0t, wr0t)


def _stage_b(p0, cnts, r0, b0, gamma, beta, wl2t, wr2t, b2):
  def body(p_ref, cnt_ref, r0_ref, b0_ref, g_ref, be_ref, wl_ref, wr_ref,
           b2_ref, y2_ref, r2b_ref, inv8_ref):
    i = pl.program_id(0)
    agg = jnp.concatenate([p_ref[0], p_ref[1]], axis=1)
    cnt = jnp.sum(cnt_ref[:, pl.ds(i * BR, BR)], axis=0)[:, None]
    inv = 1.0 / jnp.maximum(cnt, 1.0)
    pre = agg * inv + r0_ref[...] + b0_ref[...]
    h = jnp.maximum(pre * (g_ref[...] * _BN_SCALE) + be_ref[...], 0.0)
    y2 = jnp.dot(h, wl_ref[...], preferred_element_type=jnp.float32)
    y2_ref[0] = y2[:, :D2]
    y2_ref[1] = y2[:, D2:]
    r2b_ref[...] = jnp.dot(h, wr_ref[...],
                           preferred_element_type=jnp.float32) + b2_ref[...]
    inv8_ref[...] = jnp.broadcast_to(inv, (BR, 8))

  return pl.pallas_call(
      body,
      grid=(ACC_N // BR,),
      in_specs=[
          pl.BlockSpec((NC, BR, D0), lambda i: (0, i, 0)),
          pl.BlockSpec((NC * NS, ACC_N), lambda i: (0, 0)),
          pl.BlockSpec((BR, NHID), lambda i: (i, 0)),
          pl.BlockSpec((NHID,), lambda i: (0,)),
          pl.BlockSpec((NHID,), lambda i: (0,)),
          pl.BlockSpec((NHID,), lambda i: (0,)),
          pl.BlockSpec((NHID, NCLASS), lambda i: (0, 0)),
          pl.BlockSpec((NHID, NCLASS), lambda i: (0, 0)),
          pl.BlockSpec((NCLASS,), lambda i: (0,)),
      ],
      out_specs=[
          pl.BlockSpec((NC, BR, D2), lambda i: (0, i, 0)),
          pl.BlockSpec((BR, NCLASS), lambda i: (i, 0)),
          pl.BlockSpec((BR, 8), lambda i: (i, 0)),
      ],
      out_shape=[
          jax.ShapeDtypeStruct((NC, ACC_N, D2), jnp.float32),
          jax.ShapeDtypeStruct((ACC_N, NCLASS), jnp.float32),
          jax.ShapeDtypeStruct((ACC_N, 8), jnp.float32),
      ],
  )(p0, cnts, r0, b0, gamma, beta, wl2t, wr2t, b2)


def _stage_c(p2, r2b, inv8):
  def body(p_ref, r_ref, inv_ref, out_ref):
    psum = jnp.concatenate([p_ref[0], p_ref[1]], axis=1)
    out_ref[...] = psum * inv_ref[:, 0:1] + r_ref[...]

  return pl.pallas_call(
      body,
      grid=(ACC_N // BR,),
      in_specs=[
          pl.BlockSpec((NC, BR, D2), lambda i: (0, i, 0)),
          pl.BlockSpec((BR, NCLASS), lambda i: (i, 0)),
          pl.BlockSpec((BR, 8), lambda i: (i, 0)),
      ],
      out_specs=pl.BlockSpec((BR, NCLASS), lambda i: (i, 0)),
      out_shape=jax.ShapeDtypeStruct((ACC_N, NCLASS), jnp.float32),
  )(p2, r2b, inv8)


@jax.jit
def kernel(x, edge_index, Wl0, Wr0, b0, gamma, beta, Wl2, Wr2, b2):
  xp = jnp.pad(x, ((0, ACC_N - N), (0, 0)))
  src = jnp.concatenate(
      [edge_index[0], jnp.zeros((E_PAD - E,), jnp.int32)]).reshape(
          NS, NCHUNK, CH)
  src2 = jnp.stack([src, src + ACC_N])
  dst2 = jnp.concatenate(
      [edge_index[1], jnp.full((E_PAD - E,), N, jnp.int32)]).reshape(
          NS, NCHUNK, CH)

  cnts = _count_k(dst2)
  ycat, r0 = _stage_a(xp, Wl0.T, Wr0.T)
  p0 = _seg_sum_l0(ycat.reshape(NC * ACC_N, D0), src2, dst2)
  y2, r2b, inv8 = _stage_b(p0, cnts, r0, b0, gamma, beta, Wl2.T, Wr2.T, b2)
  p2 = _seg_sum_l2(y2.reshape(NC * ACC_N, D2), src2, dst2)
  out = _stage_c(p2, r2b, inv8)
  return out[:N]


# speedup vs baseline: 1.1439x; 1.1439x over previous
import functools
import math

import jax
import jax.numpy as jnp
from jax import lax
from jax.experimental import pallas as pl
from jax.experimental.pallas import tpu as pltpu
from jax.experimental.pallas import tpu_sc as plsc

N = 10000
E = 320000
NFEAT = 128
NHID = 128
NCLASS = 64
BN_EPS = 1e-5

NC = 2
NS = 16
NW = NC * NS
CH = 128
NCHUNK = 158
EPT = NCHUNK * CH
E_PAD = NS * EPT
ACC_N = 10240
D0 = 80
D2 = 32
BR = 2048
_BN_SCALE = 1.0 / math.sqrt(1.0 + BN_EPS)


def _make_seg_sum(drow):
  mesh = plsc.VectorSubcoreMesh(core_axis_name="c", subcore_axis_name="s")
  rps = ACC_N // NS

  @functools.partial(
      pl.kernel,
      out_type=jax.ShapeDtypeStruct((NC, ACC_N, drow), jnp.float32),
      mesh=mesh,
      compiler_params=pltpu.CompilerParams(use_tc_tiling_on_sc=False,
                                           needs_layout_passes=False),
      scratch_types=[
          pltpu.VMEM((NCHUNK, CH), jnp.int32),
          pltpu.VMEM((NCHUNK, CH), jnp.int32),
          pltpu.VMEM((CH, drow), jnp.float32),
          pltpu.VMEM((CH, drow), jnp.float32),
          pltpu.VMEM((16, drow), jnp.float32),
          pltpu.VMEM_SHARED((ACC_N, drow), jnp.float32),
          pltpu.SemaphoreType.DMA,
          pltpu.SemaphoreType.DMA,
      ],
  )
  def seg_sum(y_hbm, src_hbm, dst_hbm, out_hbm,
              src_v, dst_v, rows0, rows1, zbuf, acc, sem0, sem1):
    c = lax.axis_index("c")
    s = lax.axis_index("s")

    zv = jnp.zeros((16,), jnp.float32)
    for i in range(16):
      for j in range(drow // 16):
        zbuf[i, pl.ds(j * 16, 16)] = zv

    @pl.loop(0, rps // 16)
    def _zero(k):
      pltpu.sync_copy(zbuf, acc.at[pl.ds(s * rps + k * 16, 16)])

    pltpu.sync_copy(src_hbm.at[c, s], src_v)
    pltpu.sync_copy(dst_hbm.at[s], dst_v)
    plsc.subcore_barrier()

    pltpu.async_copy(y_hbm.at[src_v.at[0]], rows0, sem0)
    pltpu.async_copy(y_hbm.at[src_v.at[1]], rows1, sem1)

    @pl.loop(0, (NCHUNK - 2) // 2)
    def _chunks(j):
      i = 2 * j
      pltpu.make_async_copy(y_hbm.at[src_v.at[i]], rows0, sem0).wait()
      pltpu.sync_copy(rows0, acc.at[dst_v.at[i]], add=True)
      pltpu.async_copy(y_hbm.at[src_v.at[i + 2]], rows0, sem0)
      pltpu.make_async_copy(y_hbm.at[src_v.at[i + 1]], rows1, sem1).wait()
      pltpu.sync_copy(rows1, acc.at[dst_v.at[i + 1]], add=True)
      pltpu.async_copy(y_hbm.at[src_v.at[i + 3]], rows1, sem1)

    pltpu.make_async_copy(y_hbm.at[src_v.at[NCHUNK - 2]], rows0, sem0).wait()
    pltpu.sync_copy(rows0, acc.at[dst_v.at[NCHUNK - 2]], add=True)
    pltpu.make_async_copy(y_hbm.at[src_v.at[NCHUNK - 1]], rows1, sem1).wait()
    pltpu.sync_copy(rows1, acc.at[dst_v.at[NCHUNK - 1]], add=True)

    plsc.subcore_barrier()
    pltpu.sync_copy(acc.at[pl.ds(s * rps, rps)],
                    out_hbm.at[c, pl.ds(s * rps, rps)])

  return seg_sum


_seg_sum_l0 = _make_seg_sum(D0)
_seg_sum_l2 = _make_seg_sum(D2)


def _stage_a(x, wl0t, wr0t):
  def body(x_ref, wl_ref, wr_ref, ycat_ref, r0_ref):
    xv = x_ref[...]
    y0 = jnp.dot(xv, wl_ref[...], preferred_element_type=jnp.float32)
    ones = jnp.ones((BR, 16), jnp.float32)
    ycat_ref[0] = jnp.concatenate([y0[:, :64], ones], axis=1)
    ycat_ref[1] = jnp.concatenate([y0[:, 64:], ones], axis=1)
    r0_ref[...] = jnp.dot(xv, wr_ref[...], preferred_element_type=jnp.float32)

  return pl.pallas_call(
      body,
      grid=(ACC_N // BR,),
      in_specs=[
          pl.BlockSpec((BR, NFEAT), lambda i: (i, 0)),
          pl.BlockSpec((NFEAT, NHID), lambda i: (0, 0)),
          pl.BlockSpec((NFEAT, NHID), lambda i: (0, 0)),
      ],
      out_specs=[
          pl.BlockSpec((NC, BR, D0), lambda i: (0, i, 0)),
          pl.BlockSpec((BR, NHID), lambda i: (i, 0)),
      ],
      out_shape=[
          jax.ShapeDtypeStruct((NC, ACC_N, D0), jnp.float32),
          jax.ShapeDtypeStruct((ACC_N, NHID), jnp.float32),
      ],
  )(x, wl0t, wr0t)


def _stage_b(p0, r0, b0, gamma, beta, wl2t, wr2t, b2):
  def body(p_ref, r0_ref, b0_ref, g_ref, be_ref, wl_ref, wr_ref,
           b2_ref, y2_ref, r2b_ref, inv8_ref):
    agg = jnp.concatenate([p_ref[0, :, :64], p_ref[1, :, :64]], axis=1)
    cnt = p_ref[0, :, 64:65]
    inv = 1.0 / jnp.maximum(cnt, 1.0)
    pre = agg * inv + r0_ref[...] + b0_ref[...]
    h = jnp.maximum(pre * (g_ref[...] * _BN_SCALE) + be_ref[...], 0.0)
    y2 = jnp.dot(h, wl_ref[...], preferred_element_type=jnp.float32)
    y2_ref[0] = y2[:, :D2]
    y2_ref[1] = y2[:, D2:]
    r2b_ref[...] = jnp.dot(h, wr_ref[...],
                           preferred_element_type=jnp.float32) + b2_ref[...]
    inv8_ref[...] = jnp.broadcast_to(inv, (BR, 8))

  return pl.pallas_call(
      body,
      grid=(ACC_N // BR,),
      in_specs=[
          pl.BlockSpec((NC, BR, D0), lambda i: (0, i, 0)),
          pl.BlockSpec((BR, NHID), lambda i: (i, 0)),
          pl.BlockSpec((NHID,), lambda i: (0,)),
          pl.BlockSpec((NHID,), lambda i: (0,)),
          pl.BlockSpec((NHID,), lambda i: (0,)),
          pl.BlockSpec((NHID, NCLASS), lambda i: (0, 0)),
          pl.BlockSpec((NHID, NCLASS), lambda i: (0, 0)),
          pl.BlockSpec((NCLASS,), lambda i: (0,)),
      ],
      out_specs=[
          pl.BlockSpec((NC, BR, D2), lambda i: (0, i, 0)),
          pl.BlockSpec((BR, NCLASS), lambda i: (i, 0)),
          pl.BlockSpec((BR, 8), lambda i: (i, 0)),
      ],
      out_shape=[
          jax.ShapeDtypeStruct((NC, ACC_N, D2), jnp.float32),
          jax.ShapeDtypeStruct((ACC_N, NCLASS), jnp.float32),
          jax.ShapeDtypeStruct((ACC_N, 8), jnp.float32),
      ],
  )(p0, r0, b0, gamma, beta, wl2t, wr2t, b2)


def _stage_c(p2, r2b, inv8):
  def body(p_ref, r_ref, inv_ref, out_ref):
    psum = jnp.concatenate([p_ref[0], p_ref[1]], axis=1)
    out_ref[...] = psum * inv_ref[:, 0:1] + r_ref[...]

  return pl.pallas_call(
      body,
      grid=(ACC_N // BR,),
      in_specs=[
          pl.BlockSpec((NC, BR, D2), lambda i: (0, i, 0)),
          pl.BlockSpec((BR, NCLASS), lambda i: (i, 0)),
          pl.BlockSpec((BR, 8), lambda i: (i, 0)),
      ],
      out_specs=pl.BlockSpec((BR, NCLASS), lambda i: (i, 0)),
      out_shape=jax.ShapeDtypeStruct((ACC_N, NCLASS), jnp.float32),
  )(p2, r2b, inv8)


@jax.jit
def kernel(x, edge_index, Wl0, Wr0, b0, gamma, beta, Wl2, Wr2, b2):
  xp = jnp.pad(x, ((0, ACC_N - N), (0, 0)))
  src = jnp.concatenate(
      [edge_index[0], jnp.zeros((E_PAD - E,), jnp.int32)]).reshape(
          NS, NCHUNK, CH)
  src2 = jnp.stack([src, src + ACC_N])
  dst2 = jnp.concatenate(
      [edge_index[1], jnp.full((E_PAD - E,), N, jnp.int32)]).reshape(
          NS, NCHUNK, CH)

  ycat, r0 = _stage_a(xp, Wl0.T, Wr0.T)
  p0 = _seg_sum_l0(ycat.reshape(NC * ACC_N, D0), src2, dst2)
  y2, r2b, inv8 = _stage_b(p0, r0, b0, gamma, beta, Wl2.T, Wr2.T, b2)
  p2 = _seg_sum_l2(y2.reshape(NC * ACC_N, D2), src2, dst2)
  out = _stage_c(p2, r2b, inv8)
  return out[:N]
